# Initial kernel scaffold; baseline (speedup 1.0000x reference)
import jax
import jax.numpy as jnp
from jax.experimental import pallas as pl


def kernel(features, mask, event_indices, sparse_offsets, sparse_defaults, sparse_indices, sparse_values, W, b, code_weights):
    raise NotImplementedError("write your pallas kernel here")


# speedup vs baseline: 4.5672x; 4.5672x over previous
import functools

import jax
import jax.numpy as jnp
from jax import lax
from jax.experimental import pallas as pl
from jax.experimental.pallas import tpu as pltpu
from jax.experimental.pallas import tpu_sc as plsc

_NUM_CODES = 4096
_DIM = 768
_NTB = 8
_BATCH = 1024
_N = _BATCH * _NTB
_NEV = 16384
_LN2 = 0.6931471805599453

_TM = 512
_TN = 512

_NW = 32
_NG = _NEV + _N
_NPW = _NG // _NW


def _fulla_body(x_ref, w_ref, b_ref, o_ref):
    o_ref[...] = jnp.dot(x_ref[...], w_ref[...],
                         preferred_element_type=jnp.float32) + b_ref[...]


def _logits_body(a_ref, c_ref, d_ref, l_ref, acc_ref):
    s = lax.dot_general(a_ref[...], c_ref[...], (((1,), (1,)), ((), ())),
                        preferred_element_type=jnp.float32)
    lg = jnp.exp2(s)
    l_ref[...] = lg
    ps = lg[:, 0:128]
    for k in range(1, _TN // 128):
        ps = ps + lg[:, k * 128:(k + 1) * 128]
    part = jnp.sum(ps * jnp.exp2(d_ref[...]))

    @pl.when((pl.program_id(0) == 0) & (pl.program_id(1) == 0))
    def _():
        acc_ref[0, 0] = 0.0

    acc_ref[0, 0] += part


def _final_body(acc_ref, d_ref, v_ref, gsv_ref, gev_ref, m_ref, o_ref):
    e2d = jnp.exp2(d_ref[...])
    e2v = jnp.exp2(v_ref[...])
    corr = jnp.sum((e2v - e2d) * gsv_ref[...])
    evs = jnp.sum(jnp.log2(gev_ref[...]))
    nm = jnp.sum(m_ref[...])
    exp_mean = (acc_ref[0, 0] + corr) / (_N * _NUM_CODES)
    survival = exp_mean * (_N / nm)
    event = -_LN2 * evs / (nm * _NUM_CODES)
    o_ref[0, 0] = survival + event


def _make_sc_gather():
    mesh = plsc.VectorSubcoreMesh(core_axis_name="c", subcore_axis_name="s")

    @functools.partial(
        pl.kernel,
        out_type=jax.ShapeDtypeStruct((_NG,), jnp.float32),
        mesh=mesh,
        scratch_types=[
            pltpu.VMEM((_NPW,), jnp.int32),
            pltpu.VMEM((_NPW,), jnp.float32),
            pltpu.SemaphoreType.DMA,
        ],
    )
    def gather(table_hbm, idx_hbm, out_hbm, idx_v, vals_v, sem):
        wid = lax.axis_index("s") * 2 + lax.axis_index("c")
        base = wid * _NPW
        pltpu.sync_copy(idx_hbm.at[pl.ds(base, _NPW)], idx_v)
        pltpu.async_copy(table_hbm.at[idx_v], vals_v, sem).wait()
        pltpu.sync_copy(vals_v, out_hbm.at[pl.ds(base, _NPW)])

    return gather


_sc_gather = _make_sc_gather()


def kernel(features, mask, event_indices, sparse_offsets, sparse_defaults,
           sparse_indices, sparse_values, W, b, code_weights):
    f32 = jnp.float32

    Wp = jnp.pad(W.reshape(_DIM, _NTB, _DIM - 1),
                 ((0, 0), (0, 0), (0, 1))).reshape(_DIM, _NTB * _DIM)
    bp = jnp.pad(b.reshape(_NTB, _DIM - 1), ((0, 0), (0, 1)),
                 constant_values=1.0).reshape(1, _NTB * _DIM)

    fulla2 = pl.pallas_call(
        _fulla_body,
        grid=(8,),
        in_specs=[
            pl.BlockSpec((_BATCH, _DIM), lambda j: (0, 0)),
            pl.BlockSpec((_DIM, _DIM), lambda j: (0, j)),
            pl.BlockSpec((1, _DIM), lambda j: (0, j)),
        ],
        out_specs=pl.BlockSpec((_BATCH, _DIM), lambda j: (0, j)),
        out_shape=jax.ShapeDtypeStruct((_BATCH, _NTB * _DIM), f32),
    )(features, Wp, bp)
    full_a = fulla2.reshape(_N, _DIM)

    d_col = sparse_defaults.reshape(_N, 1)
    logits, acc = pl.pallas_call(
        _logits_body,
        grid=(_N // _TM, _NUM_CODES // _TN),
        in_specs=[
            pl.BlockSpec((_TM, _DIM), lambda i, j: (i, 0)),
            pl.BlockSpec((_TN, _DIM), lambda i, j: (j, 0)),
            pl.BlockSpec((_TM, 1), lambda i, j: (i, 0)),
        ],
        out_specs=[
            pl.BlockSpec((_TM, _TN), lambda i, j: (i, j)),
            pl.BlockSpec((1, 1), lambda i, j: (0, 0),
                         memory_space=pltpu.SMEM),
        ],
        out_shape=[
            jax.ShapeDtypeStruct((_N, _NUM_CODES), f32),
            jax.ShapeDtypeStruct((1, 1), f32),
        ],
    )(full_a, code_weights, d_col)

    ev_flat = (event_indices[:, 0].astype(jnp.int32) * _NUM_CODES
               + event_indices[:, 1].astype(jnp.int32))
    sv_flat = (jnp.arange(_N, dtype=jnp.int32) * _NUM_CODES
               + sparse_indices.astype(jnp.int32))
    flat = jnp.concatenate([ev_flat, sv_flat])
    table = logits.reshape(_N * _NUM_CODES)
    gathered = _sc_gather(table, flat)

    gev = gathered[:_NEV].reshape(128, 128)
    gsv = gathered[_NEV:].reshape(64, 128)
    d2 = sparse_defaults.reshape(64, 128)
    v2 = sparse_values.reshape(64, 128)
    m2 = mask.astype(f32).reshape(8, 128)

    loss = pl.pallas_call(
        _final_body,
        in_specs=[
            pl.BlockSpec(memory_space=pltpu.SMEM),
            pl.BlockSpec((64, 128), lambda: (0, 0)),
            pl.BlockSpec((64, 128), lambda: (0, 0)),
            pl.BlockSpec((64, 128), lambda: (0, 0)),
            pl.BlockSpec((128, 128), lambda: (0, 0)),
            pl.BlockSpec((8, 128), lambda: (0, 0)),
        ],
        out_specs=pl.BlockSpec(memory_space=pltpu.SMEM),
        out_shape=jax.ShapeDtypeStruct((1, 1), f32),
    )(acc, d2, v2, gsv, gev, m2)

    return loss[0, 0], logits


# speedup vs baseline: 5.4455x; 1.1923x over previous
import functools

import jax
import jax.numpy as jnp
from jax import lax
from jax.experimental import pallas as pl
from jax.experimental.pallas import tpu as pltpu
from jax.experimental.pallas import tpu_sc as plsc

_NUM_CODES = 4096
_DIM = 768
_NTB = 8
_BATCH = 1024
_N = _BATCH * _NTB
_NEV = 16384
_LN2 = 0.6931471805599453

_TM = 1024
_TN = 1024

_NW = 32
_NG = _NEV + _N
_NPW = _NG // _NW


def _fulla_body(x_ref, w_ref, b_ref, o_ref):
    x = x_ref[...]
    w = w_ref[...]
    xh = x.astype(jnp.bfloat16)
    xl = (x - xh.astype(jnp.float32)).astype(jnp.bfloat16)
    wh = w.astype(jnp.bfloat16)
    wl = (w - wh.astype(jnp.float32)).astype(jnp.bfloat16)
    dn = (((1,), (0,)), ((), ()))
    o = lax.dot_general(xh, wh, dn, preferred_element_type=jnp.float32)
    o += lax.dot_general(xl, wh, dn, preferred_element_type=jnp.float32)
    o += lax.dot_general(xh, wl, dn, preferred_element_type=jnp.float32)
    o_ref[...] = o + b_ref[...]


def _logits_body(a_ref, c_ref, d_ref, l_ref, acc_ref):
    s = lax.dot_general(a_ref[...].astype(jnp.bfloat16),
                        c_ref[...].astype(jnp.bfloat16),
                        (((1,), (1,)), ((), ())),
                        preferred_element_type=jnp.float32)
    lg = jnp.exp2(s)
    l_ref[...] = lg
    ps = lg[:, 0:128]
    for k in range(1, _TN // 128):
        ps = ps + lg[:, k * 128:(k + 1) * 128]
    part = jnp.sum(ps * jnp.exp2(d_ref[...]))

    @pl.when((pl.program_id(0) == 0) & (pl.program_id(1) == 0))
    def _():
        acc_ref[0, 0] = 0.0

    acc_ref[0, 0] += part


def _final_body(acc_ref, d_ref, v_ref, gsv_ref, gev_ref, m_ref, o_ref):
    e2d = jnp.exp2(d_ref[...])
    e2v = jnp.exp2(v_ref[...])
    corr = jnp.sum((e2v - e2d) * gsv_ref[...])
    evs = jnp.sum(jnp.log2(gev_ref[...]))
    nm = jnp.sum(m_ref[...])
    exp_mean = (acc_ref[0, 0] + corr) / (_N * _NUM_CODES)
    survival = exp_mean * (_N / nm)
    event = -_LN2 * evs / (nm * _NUM_CODES)
    o_ref[0, 0] = survival + event


def _make_sc_gather():
    mesh = plsc.VectorSubcoreMesh(core_axis_name="c", subcore_axis_name="s")

    @functools.partial(
        pl.kernel,
        out_type=jax.ShapeDtypeStruct((_NG,), jnp.float32),
        mesh=mesh,
        scratch_types=[
            pltpu.VMEM((_NPW,), jnp.int32),
            pltpu.VMEM((_NPW,), jnp.float32),
            pltpu.SemaphoreType.DMA,
        ],
    )
    def gather(table_hbm, idx_hbm, out_hbm, idx_v, vals_v, sem):
        wid = lax.axis_index("s") * 2 + lax.axis_index("c")
        base = wid * _NPW
        pltpu.sync_copy(idx_hbm.at[pl.ds(base, _NPW)], idx_v)
        pltpu.async_copy(table_hbm.at[idx_v], vals_v, sem).wait()
        pltpu.sync_copy(vals_v, out_hbm.at[pl.ds(base, _NPW)])

    return gather


_sc_gather = _make_sc_gather()


def kernel(features, mask, event_indices, sparse_offsets, sparse_defaults,
           sparse_indices, sparse_values, W, b, code_weights):
    f32 = jnp.float32

    Wp = jnp.pad(W.reshape(_DIM, _NTB, _DIM - 1),
                 ((0, 0), (0, 0), (0, 1))).reshape(_DIM, _NTB * _DIM)
    bp = jnp.pad(b.reshape(_NTB, _DIM - 1), ((0, 0), (0, 1)),
                 constant_values=1.0).reshape(1, _NTB * _DIM)

    fulla2 = pl.pallas_call(
        _fulla_body,
        grid=(8,),
        in_specs=[
            pl.BlockSpec((_BATCH, _DIM), lambda j: (0, 0)),
            pl.BlockSpec((_DIM, _DIM), lambda j: (0, j)),
            pl.BlockSpec((1, _DIM), lambda j: (0, j)),
        ],
        out_specs=pl.BlockSpec((_BATCH, _DIM), lambda j: (0, j)),
        out_shape=jax.ShapeDtypeStruct((_BATCH, _NTB * _DIM), f32),
    )(features, Wp, bp)
    full_a = fulla2.reshape(_N, _DIM)

    d_col = sparse_defaults.reshape(_N, 1)
    logits, acc = pl.pallas_call(
        _logits_body,
        grid=(_N // _TM, _NUM_CODES // _TN),
        in_specs=[
            pl.BlockSpec((_TM, _DIM), lambda i, j: (i, 0)),
            pl.BlockSpec((_TN, _DIM), lambda i, j: (j, 0)),
            pl.BlockSpec((_TM, 1), lambda i, j: (i, 0)),
        ],
        out_specs=[
            pl.BlockSpec((_TM, _TN), lambda i, j: (i, j)),
            pl.BlockSpec((1, 1), lambda i, j: (0, 0),
                         memory_space=pltpu.SMEM),
        ],
        out_shape=[
            jax.ShapeDtypeStruct((_N, _NUM_CODES), f32),
            jax.ShapeDtypeStruct((1, 1), f32),
        ],
    )(full_a, code_weights, d_col)

    ev_flat = (event_indices[:, 0].astype(jnp.int32) * _NUM_CODES
               + event_indices[:, 1].astype(jnp.int32))
    sv_flat = (jnp.arange(_N, dtype=jnp.int32) * _NUM_CODES
               + sparse_indices.astype(jnp.int32))
    flat = jnp.concatenate([ev_flat, sv_flat])
    table = logits.reshape(_N * _NUM_CODES)
    gathered = _sc_gather(table, flat)

    gev = gathered[:_NEV].reshape(128, 128)
    gsv = gathered[_NEV:].reshape(64, 128)
    d2 = sparse_defaults.reshape(64, 128)
    v2 = sparse_values.reshape(64, 128)
    m2 = mask.astype(f32).reshape(8, 128)

    loss = pl.pallas_call(
        _final_body,
        in_specs=[
            pl.BlockSpec(memory_space=pltpu.SMEM),
            pl.BlockSpec((64, 128), lambda: (0, 0)),
            pl.BlockSpec((64, 128), lambda: (0, 0)),
            pl.BlockSpec((64, 128), lambda: (0, 0)),
            pl.BlockSpec((128, 128), lambda: (0, 0)),
            pl.BlockSpec((8, 128), lambda: (0, 0)),
        ],
        out_specs=pl.BlockSpec(memory_space=pltpu.SMEM),
        out_shape=jax.ShapeDtypeStruct((1, 1), f32),
    )(acc, d2, v2, gsv, gev, m2)

    return loss[0, 0], logits


# speedup vs baseline: 6.4550x; 1.1854x over previous
import functools

import jax
import jax.numpy as jnp
from jax import lax
from jax.experimental import pallas as pl
from jax.experimental.pallas import tpu as pltpu
from jax.experimental.pallas import tpu_sc as plsc

_NUM_CODES = 4096
_DIM = 768
_NTB = 8
_BATCH = 1024
_N = _BATCH * _NTB
_NEV = 16384
_LN2 = 0.6931471805599453

_TM = 1024
_TN = 1024

_NW = 32
_NG = _NEV + _N
_NPW = _NG // _NW


def _fulla_body(x_ref, w_ref, b_ref, o_ref):
    x = x_ref[...]
    w = w_ref[...]
    xh = x.astype(jnp.bfloat16)
    xl = (x - xh.astype(jnp.float32)).astype(jnp.bfloat16)
    wh = w.astype(jnp.bfloat16)
    wl = (w - wh.astype(jnp.float32)).astype(jnp.bfloat16)
    dn = (((1,), (0,)), ((), ()))
    o = lax.dot_general(xh, wh, dn, preferred_element_type=jnp.float32)
    o += lax.dot_general(xl, wh, dn, preferred_element_type=jnp.float32)
    o += lax.dot_general(xh, wl, dn, preferred_element_type=jnp.float32)
    o_ref[...] = o + b_ref[...]


def _logits_body(a_ref, c_ref, d_ref, l_ref, acc_ref):
    s = lax.dot_general(a_ref[...].astype(jnp.bfloat16),
                        c_ref[...].astype(jnp.bfloat16),
                        (((1,), (1,)), ((), ())),
                        preferred_element_type=jnp.float32)
    lg = jnp.exp2(s)
    l_ref[...] = lg
    ps = lg[:, 0:128]
    for k in range(1, _TN // 128):
        ps = ps + lg[:, k * 128:(k + 1) * 128]
    part = jnp.sum(ps * jnp.exp2(d_ref[...]))

    @pl.when((pl.program_id(0) == 0) & (pl.program_id(1) == 0))
    def _():
        acc_ref[0, 0] = 0.0

    acc_ref[0, 0] += part


def _final_body(acc_ref, d_ref, v_ref, gsv_ref, gev_ref, m_ref, o_ref):
    e2d = jnp.exp2(d_ref[...])
    e2v = jnp.exp2(v_ref[...])
    ssv = jnp.sum(gsv_ref[...], axis=1, keepdims=True)
    corr = jnp.sum((e2v - e2d) * jnp.exp2(ssv))
    evs = jnp.sum(gev_ref[...])
    nm = jnp.sum(m_ref[...])
    exp_mean = (acc_ref[0, 0] + corr) / (_N * _NUM_CODES)
    survival = exp_mean * (_N / nm)
    event = -_LN2 * evs / (nm * _NUM_CODES)
    o_ref[0, 0] = survival + event


_KB = 32
_EV_PW = _NEV // _NW
_SV_PW = _N // _NW


def _make_sc_dots():
    mesh = plsc.VectorSubcoreMesh(core_axis_name="c", subcore_axis_name="s")

    @functools.partial(
        pl.kernel,
        out_type=(jax.ShapeDtypeStruct((_NW, 16), jnp.float32),
                  jax.ShapeDtypeStruct((_N, 16), jnp.float32)),
        mesh=mesh,
        scratch_types=[
            pltpu.VMEM((_EV_PW,), jnp.int32),
            pltpu.VMEM((_EV_PW,), jnp.int32),
            pltpu.VMEM((_KB, _DIM), jnp.float32),
            pltpu.VMEM((_KB, _DIM), jnp.float32),
            pltpu.VMEM((_SV_PW, 16), jnp.float32),
            pltpu.VMEM((16,), jnp.float32),
            pltpu.SemaphoreType.DMA,
            pltpu.SemaphoreType.DMA,
        ],
    )
    def dots(fa_hbm, cw_hbm, r_hbm, c_hbm, ev_hbm, sv_hbm,
             r_v, c_v, fa_v, cw_v, sv_v, ev_v, sem_a, sem_b):
        wid = lax.axis_index("s") * 2 + lax.axis_index("c")

        def dot16(p):
            acc = fa_v[p, pl.ds(0, 16)] * cw_v[p, pl.ds(0, 16)]
            for k in range(1, _DIM // 16):
                acc = acc + (fa_v[p, pl.ds(k * 16, 16)]
                             * cw_v[p, pl.ds(k * 16, 16)])
            return acc

        base_e = wid * _EV_PW
        pltpu.sync_copy(r_hbm.at[pl.ds(base_e, _EV_PW)], r_v)
        pltpu.sync_copy(c_hbm.at[pl.ds(base_e, _EV_PW)], c_v)

        def batch_e(bi, eacc):
            off = bi * _KB
            cp_a = pltpu.async_copy(fa_hbm.at[r_v.at[pl.ds(off, _KB)]],
                                    fa_v, sem_a)
            cp_b = pltpu.async_copy(cw_hbm.at[c_v.at[pl.ds(off, _KB)]],
                                    cw_v, sem_b)
            cp_a.wait()
            cp_b.wait()
            return lax.fori_loop(0, _KB, lambda p, a: a + dot16(p), eacc)

        eacc = lax.fori_loop(0, _EV_PW // _KB, batch_e,
                             jnp.zeros((16,), jnp.float32))
        ev_v[...] = eacc
        pltpu.sync_copy(ev_v, ev_hbm.at[wid])

        base_s = _NEV + wid * _SV_PW
        pltpu.sync_copy(r_hbm.at[pl.ds(base_s, _SV_PW)],
                        r_v.at[pl.ds(0, _SV_PW)])
        pltpu.sync_copy(c_hbm.at[pl.ds(base_s, _SV_PW)],
                        c_v.at[pl.ds(0, _SV_PW)])

        def batch_s(bi, carry):
            off = bi * _KB
            cp_a = pltpu.async_copy(fa_hbm.at[r_v.at[pl.ds(off, _KB)]],
                                    fa_v, sem_a)
            cp_b = pltpu.async_copy(cw_hbm.at[c_v.at[pl.ds(off, _KB)]],
                                    cw_v, sem_b)
            cp_a.wait()
            cp_b.wait()

            def pair(p, carry2):
                sv_v[off + p] = dot16(p)
                return carry2

            lax.fori_loop(0, _KB, pair, 0)
            return carry

        lax.fori_loop(0, _SV_PW // _KB, batch_s, 0)
        pltpu.sync_copy(sv_v, sv_hbm.at[pl.ds(wid * _SV_PW, _SV_PW)])

    return dots


_sc_dots = _make_sc_dots()


def kernel(features, mask, event_indices, sparse_offsets, sparse_defaults,
           sparse_indices, sparse_values, W, b, code_weights):
    f32 = jnp.float32

    Wp = jnp.pad(W.reshape(_DIM, _NTB, _DIM - 1),
                 ((0, 0), (0, 0), (0, 1))).reshape(_DIM, _NTB * _DIM)
    bp = jnp.pad(b.reshape(_NTB, _DIM - 1), ((0, 0), (0, 1)),
                 constant_values=1.0).reshape(1, _NTB * _DIM)

    fulla2 = pl.pallas_call(
        _fulla_body,
        grid=(8,),
        in_specs=[
            pl.BlockSpec((_BATCH, _DIM), lambda j: (0, 0)),
            pl.BlockSpec((_DIM, _DIM), lambda j: (0, j)),
            pl.BlockSpec((1, _DIM), lambda j: (0, j)),
        ],
        out_specs=pl.BlockSpec((_BATCH, _DIM), lambda j: (0, j)),
        out_shape=jax.ShapeDtypeStruct((_BATCH, _NTB * _DIM), f32),
    )(features, Wp, bp)
    full_a = fulla2.reshape(_N, _DIM)

    d_col = sparse_defaults.reshape(_N, 1)
    logits, acc = pl.pallas_call(
        _logits_body,
        grid=(_N // _TM, _NUM_CODES // _TN),
        in_specs=[
            pl.BlockSpec((_TM, _DIM), lambda i, j: (i, 0)),
            pl.BlockSpec((_TN, _DIM), lambda i, j: (j, 0)),
            pl.BlockSpec((_TM, 1), lambda i, j: (i, 0)),
        ],
        out_specs=[
            pl.BlockSpec((_TM, _TN), lambda i, j: (i, j)),
            pl.BlockSpec((1, 1), lambda i, j: (0, 0),
                         memory_space=pltpu.SMEM),
        ],
        out_shape=[
            jax.ShapeDtypeStruct((_N, _NUM_CODES), f32),
            jax.ShapeDtypeStruct((1, 1), f32),
        ],
    )(full_a, code_weights, d_col)

    r_all = jnp.concatenate([event_indices[:, 0].astype(jnp.int32),
                             jnp.arange(_N, dtype=jnp.int32)])
    c_all = jnp.concatenate([event_indices[:, 1].astype(jnp.int32),
                             sparse_indices.astype(jnp.int32)])
    gev, gsv = _sc_dots(full_a, code_weights, r_all, c_all)
    v_col = sparse_values.reshape(_N, 1)
    m2 = mask.astype(f32).reshape(8, 128)

    loss = pl.pallas_call(
        _final_body,
        in_specs=[
            pl.BlockSpec(memory_space=pltpu.SMEM),
            pl.BlockSpec((_N, 1), lambda: (0, 0)),
            pl.BlockSpec((_N, 1), lambda: (0, 0)),
            pl.BlockSpec((_N, 16), lambda: (0, 0)),
            pl.BlockSpec((_NW, 16), lambda: (0, 0)),
            pl.BlockSpec((8, 128), lambda: (0, 0)),
        ],
        out_specs=pl.BlockSpec(memory_space=pltpu.SMEM),
        out_shape=jax.ShapeDtypeStruct((1, 1), f32),
    )(acc, d_col, v_col, gsv, gev, m2)

    return loss[0, 0], logits


# speedup vs baseline: 7.4949x; 1.1611x over previous
import functools

import jax
import jax.numpy as jnp
from jax import lax
from jax.experimental import pallas as pl
from jax.experimental.pallas import tpu as pltpu
from jax.experimental.pallas import tpu_sc as plsc

_NUM_CODES = 4096
_DIM = 768
_NTB = 8
_BATCH = 1024
_N = _BATCH * _NTB
_NEV = 16384
_LN2 = 0.6931471805599453

_TM = 1024
_TN = 1024

_NW = 32
_NG = _NEV + _N
_NPW = _NG // _NW


def _fulla_body(x_ref, w_ref, b_ref, o_ref):
    x = x_ref[...]
    xh = x.astype(jnp.bfloat16)
    xl = (x - xh.astype(jnp.float32)).astype(jnp.bfloat16)
    dn = (((1,), (0,)), ((), ()))
    ones = jnp.ones((x.shape[0], 1), jnp.float32)
    for t in range(_NTB):
        w = w_ref[:, pl.ds(t * (_DIM - 1), _DIM - 1)]
        wh = w.astype(jnp.bfloat16)
        wl = (w - wh.astype(jnp.float32)).astype(jnp.bfloat16)
        o = lax.dot_general(xh, wh, dn, preferred_element_type=jnp.float32)
        o += lax.dot_general(xl, wh, dn, preferred_element_type=jnp.float32)
        o += lax.dot_general(xh, wl, dn, preferred_element_type=jnp.float32)
        o = o + b_ref[:, pl.ds(t * (_DIM - 1), _DIM - 1)]
        o_ref[:, t, :] = jnp.concatenate([o, ones], axis=1)


def _logits_body(a_ref, c_ref, d_ref, l_ref, acc_ref):
    s = lax.dot_general(a_ref[...].astype(jnp.bfloat16),
                        c_ref[...].astype(jnp.bfloat16),
                        (((1,), (1,)), ((), ())),
                        preferred_element_type=jnp.float32)
    lg = jnp.exp2(s)
    l_ref[...] = lg
    ps = lg[:, 0:128]
    for k in range(1, _TN // 128):
        ps = ps + lg[:, k * 128:(k + 1) * 128]
    part = jnp.sum(ps * jnp.exp2(d_ref[...]))

    @pl.when((pl.program_id(0) == 0) & (pl.program_id(1) == 0))
    def _():
        acc_ref[0, 0] = 0.0

    acc_ref[0, 0] += part


def _final_body(acc_ref, d_ref, v_ref, gsv_ref, gev_ref, m_ref, o_ref):
    e2d = jnp.exp2(d_ref[...])
    e2v = jnp.exp2(v_ref[...])
    ssv = jnp.sum(gsv_ref[...], axis=1, keepdims=True)
    corr = jnp.sum((e2v - e2d) * jnp.exp2(ssv))
    evs = jnp.sum(gev_ref[...])
    nm = jnp.sum(m_ref[...])
    exp_mean = (acc_ref[0, 0] + corr) / (_N * _NUM_CODES)
    survival = exp_mean * (_N / nm)
    event = -_LN2 * evs / (nm * _NUM_CODES)
    o_ref[0, 0] = survival + event


_KB = 32
_EV_PW = _NEV // _NW
_SV_PW = _N // _NW


def _make_sc_dots():
    mesh = plsc.VectorSubcoreMesh(core_axis_name="c", subcore_axis_name="s")

    @functools.partial(
        pl.kernel,
        out_type=(jax.ShapeDtypeStruct((_NW, 16), jnp.float32),
                  jax.ShapeDtypeStruct((_N, 16), jnp.float32)),
        mesh=mesh,
        scratch_types=[
            pltpu.VMEM((_EV_PW,), jnp.int32),
            pltpu.VMEM((_EV_PW,), jnp.int32),
            pltpu.VMEM((_KB, _DIM), jnp.float32),
            pltpu.VMEM((_KB, _DIM), jnp.float32),
            pltpu.VMEM((_SV_PW, 16), jnp.float32),
            pltpu.VMEM((16,), jnp.float32),
            pltpu.SemaphoreType.DMA,
            pltpu.SemaphoreType.DMA,
        ],
    )
    def dots(fa_hbm, cw_hbm, r_hbm, c_hbm, ev_hbm, sv_hbm,
             r_v, c_v, fa_v, cw_v, sv_v, ev_v, sem_a, sem_b):
        wid = lax.axis_index("s") * 2 + lax.axis_index("c")

        def dot16(p):
            acc = fa_v[p, pl.ds(0, 16)] * cw_v[p, pl.ds(0, 16)]
            for k in range(1, _DIM // 16):
                acc = acc + (fa_v[p, pl.ds(k * 16, 16)]
                             * cw_v[p, pl.ds(k * 16, 16)])
            return acc

        base_e = wid * _EV_PW
        pltpu.sync_copy(r_hbm.at[pl.ds(base_e, _EV_PW)], r_v)
        pltpu.sync_copy(c_hbm.at[pl.ds(base_e, _EV_PW)], c_v)

        def batch_e(bi, eacc):
            off = bi * _KB
            cp_a = pltpu.async_copy(fa_hbm.at[r_v.at[pl.ds(off, _KB)]],
                                    fa_v, sem_a)
            cp_b = pltpu.async_copy(cw_hbm.at[c_v.at[pl.ds(off, _KB)]],
                                    cw_v, sem_b)
            cp_a.wait()
            cp_b.wait()
            return lax.fori_loop(0, _KB, lambda p, a: a + dot16(p), eacc)

        eacc = lax.fori_loop(0, _EV_PW // _KB, batch_e,
                             jnp.zeros((16,), jnp.float32))
        ev_v[...] = eacc
        pltpu.sync_copy(ev_v, ev_hbm.at[wid])

        base_s = _NEV + wid * _SV_PW
        pltpu.sync_copy(r_hbm.at[pl.ds(base_s, _SV_PW)],
                        r_v.at[pl.ds(0, _SV_PW)])
        pltpu.sync_copy(c_hbm.at[pl.ds(base_s, _SV_PW)],
                        c_v.at[pl.ds(0, _SV_PW)])

        def batch_s(bi, carry):
            off = bi * _KB
            cp_a = pltpu.async_copy(fa_hbm.at[r_v.at[pl.ds(off, _KB)]],
                                    fa_v, sem_a)
            cp_b = pltpu.async_copy(cw_hbm.at[c_v.at[pl.ds(off, _KB)]],
                                    cw_v, sem_b)
            cp_a.wait()
            cp_b.wait()

            def pair(p, carry2):
                sv_v[off + p] = dot16(p)
                return carry2

            lax.fori_loop(0, _KB, pair, 0)
            return carry

        lax.fori_loop(0, _SV_PW // _KB, batch_s, 0)
        pltpu.sync_copy(sv_v, sv_hbm.at[pl.ds(wid * _SV_PW, _SV_PW)])

    return dots


_sc_dots = _make_sc_dots()


def kernel(features, mask, event_indices, sparse_offsets, sparse_defaults,
           sparse_indices, sparse_values, W, b, code_weights):
    f32 = jnp.float32

    _BM = 256
    fulla3 = pl.pallas_call(
        _fulla_body,
        grid=(_BATCH // _BM,),
        in_specs=[
            pl.BlockSpec((_BM, _DIM), lambda i: (i, 0)),
            pl.BlockSpec((_DIM, _NTB * (_DIM - 1)), lambda i: (0, 0)),
            pl.BlockSpec((1, _NTB * (_DIM - 1)), lambda i: (0, 0)),
        ],
        out_specs=pl.BlockSpec((_BM, _NTB, _DIM), lambda i: (i, 0, 0)),
        out_shape=jax.ShapeDtypeStruct((_BATCH, _NTB, _DIM), f32),
    )(features, W, b.reshape(1, _NTB * (_DIM - 1)))
    full_a = fulla3.reshape(_N, _DIM)

    d_col = sparse_defaults.reshape(_N, 1)
    logits, acc = pl.pallas_call(
        _logits_body,
        grid=(_N // _TM, _NUM_CODES // _TN),
        in_specs=[
            pl.BlockSpec((_TM, _DIM), lambda i, j: (i, 0)),
            pl.BlockSpec((_TN, _DIM), lambda i, j: (j, 0)),
            pl.BlockSpec((_TM, 1), lambda i, j: (i, 0)),
        ],
        out_specs=[
            pl.BlockSpec((_TM, _TN), lambda i, j: (i, j)),
            pl.BlockSpec((1, 1), lambda i, j: (0, 0),
                         memory_space=pltpu.SMEM),
        ],
        out_shape=[
            jax.ShapeDtypeStruct((_N, _NUM_CODES), f32),
            jax.ShapeDtypeStruct((1, 1), f32),
        ],
    )(full_a, code_weights, d_col)

    r_all = jnp.concatenate([event_indices[:, 0].astype(jnp.int32),
                             jnp.arange(_N, dtype=jnp.int32)])
    c_all = jnp.concatenate([event_indices[:, 1].astype(jnp.int32),
                             sparse_indices.astype(jnp.int32)])
    gev, gsv = _sc_dots(full_a, code_weights, r_all, c_all)
    v_col = sparse_values.reshape(_N, 1)
    m2 = mask.astype(f32).reshape(8, 128)

    loss = pl.pallas_call(
        _final_body,
        in_specs=[
            pl.BlockSpec(memory_space=pltpu.SMEM),
            pl.BlockSpec((_N, 1), lambda: (0, 0)),
            pl.BlockSpec((_N, 1), lambda: (0, 0)),
            pl.BlockSpec((_N, 16), lambda: (0, 0)),
            pl.BlockSpec((_NW, 16), lambda: (0, 0)),
            pl.BlockSpec((8, 128), lambda: (0, 0)),
        ],
        out_specs=pl.BlockSpec(memory_space=pltpu.SMEM),
        out_shape=jax.ShapeDtypeStruct((1, 1), f32),
    )(acc, d_col, v_col, gsv, gev, m2)

    return loss[0, 0], logits


# speedup vs baseline: 8.2605x; 1.1022x over previous
import functools

import jax
import jax.numpy as jnp
from jax import lax
from jax.experimental import pallas as pl
from jax.experimental.pallas import tpu as pltpu
from jax.experimental.pallas import tpu_sc as plsc

_NUM_CODES = 4096
_DIM = 768
_NTB = 8
_BATCH = 1024
_N = _BATCH * _NTB
_NEV = 16384
_LN2 = 0.6931471805599453

_TM = 2048
_TN = 1024

_NW = 32
_NG = _NEV + _N
_NPW = _NG // _NW


def _fulla_body(x_ref, w_ref, b_ref, o_ref):
    xh = x_ref[...].astype(jnp.bfloat16)
    dn = (((1,), (0,)), ((), ()))
    ones = jnp.ones((xh.shape[0], 1), jnp.float32)
    for t in range(_NTB):
        wh = w_ref[:, pl.ds(t * (_DIM - 1), _DIM - 1)].astype(jnp.bfloat16)
        o = lax.dot_general(xh, wh, dn, preferred_element_type=jnp.float32)
        o = o + b_ref[:, pl.ds(t * (_DIM - 1), _DIM - 1)]
        o_ref[:, t, :] = jnp.concatenate([o, ones], axis=1)


def _logits_body(a_ref, c_ref, d8_ref, l_ref, p8_ref):
    s = lax.dot_general(a_ref[...].astype(jnp.bfloat16),
                        c_ref[...].astype(jnp.bfloat16),
                        (((1,), (1,)), ((), ())),
                        preferred_element_type=jnp.float32)
    lg = jnp.exp2(s)
    l_ref[...] = lg
    e2d8 = jnp.exp2(d8_ref[...])
    p8_ref[...] = lax.dot_general(e2d8, lg, (((1,), (0,)), ((), ())),
                                  preferred_element_type=jnp.float32)


def _final_body(p8_ref, d_ref, v_ref, gsv_ref, gev_ref, m_ref, o_ref):
    acc = jnp.sum(p8_ref[...]) * 0.125
    e2d = jnp.exp2(d_ref[...])
    e2v = jnp.exp2(v_ref[...])
    ssv = jnp.sum(gsv_ref[...], axis=1, keepdims=True)
    corr = jnp.sum((e2v - e2d) * jnp.exp2(ssv))
    evs = jnp.sum(gev_ref[...])
    nm = jnp.sum(m_ref[...])
    exp_mean = (acc + corr) / (_N * _NUM_CODES)
    survival = exp_mean * (_N / nm)
    event = -_LN2 * evs / (nm * _NUM_CODES)
    o_ref[0, 0] = survival + event


_KB = 32
_EV_PW = _NEV // _NW
_SV_PW = _N // _NW


def _make_sc_dots():
    mesh = plsc.VectorSubcoreMesh(core_axis_name="c", subcore_axis_name="s")

    @functools.partial(
        pl.kernel,
        out_type=(jax.ShapeDtypeStruct((_NW, 16), jnp.float32),
                  jax.ShapeDtypeStruct((_N, 16), jnp.float32)),
        mesh=mesh,
        scratch_types=[
            pltpu.VMEM((_EV_PW,), jnp.int32),
            pltpu.VMEM((_EV_PW,), jnp.int32),
            pltpu.VMEM((_KB, _DIM), jnp.float32),
            pltpu.VMEM((_KB, _DIM), jnp.float32),
            pltpu.VMEM((_SV_PW, 16), jnp.float32),
            pltpu.VMEM((16,), jnp.float32),
            pltpu.SemaphoreType.DMA,
            pltpu.SemaphoreType.DMA,
        ],
    )
    def dots(fa_hbm, cw_hbm, r_hbm, c_hbm, ev_hbm, sv_hbm,
             r_v, c_v, fa_v, cw_v, sv_v, ev_v, sem_a, sem_b):
        wid = lax.axis_index("s") * 2 + lax.axis_index("c")

        def dot16(p):
            acc = fa_v[p, pl.ds(0, 16)] * cw_v[p, pl.ds(0, 16)]
            for k in range(1, _DIM // 16):
                acc = acc + (fa_v[p, pl.ds(k * 16, 16)]
                             * cw_v[p, pl.ds(k * 16, 16)])
            return acc

        base_e = wid * _EV_PW
        pltpu.sync_copy(r_hbm.at[pl.ds(base_e, _EV_PW)], r_v)
        pltpu.sync_copy(c_hbm.at[pl.ds(base_e, _EV_PW)], c_v)

        def batch_e(bi, eacc):
            off = bi * _KB
            cp_a = pltpu.async_copy(fa_hbm.at[r_v.at[pl.ds(off, _KB)]],
                                    fa_v, sem_a)
            cp_b = pltpu.async_copy(cw_hbm.at[c_v.at[pl.ds(off, _KB)]],
                                    cw_v, sem_b)
            cp_a.wait()
            cp_b.wait()
            return lax.fori_loop(0, _KB, lambda p, a: a + dot16(p), eacc)

        eacc = lax.fori_loop(0, _EV_PW // _KB, batch_e,
                             jnp.zeros((16,), jnp.float32))
        ev_v[...] = eacc
        pltpu.sync_copy(ev_v, ev_hbm.at[wid])

        base_s = _NEV + wid * _SV_PW
        pltpu.sync_copy(r_hbm.at[pl.ds(base_s, _SV_PW)],
                        r_v.at[pl.ds(0, _SV_PW)])
        pltpu.sync_copy(c_hbm.at[pl.ds(base_s, _SV_PW)],
                        c_v.at[pl.ds(0, _SV_PW)])

        def batch_s(bi, carry):
            off = bi * _KB
            cp_a = pltpu.async_copy(fa_hbm.at[r_v.at[pl.ds(off, _KB)]],
                                    fa_v, sem_a)
            cp_b = pltpu.async_copy(cw_hbm.at[c_v.at[pl.ds(off, _KB)]],
                                    cw_v, sem_b)
            cp_a.wait()
            cp_b.wait()

            def pair(p, carry2):
                sv_v[off + p] = dot16(p)
                return carry2

            lax.fori_loop(0, _KB, pair, 0)
            return carry

        lax.fori_loop(0, _SV_PW // _KB, batch_s, 0)
        pltpu.sync_copy(sv_v, sv_hbm.at[pl.ds(wid * _SV_PW, _SV_PW)])

    return dots


_sc_dots = _make_sc_dots()


def kernel(features, mask, event_indices, sparse_offsets, sparse_defaults,
           sparse_indices, sparse_values, W, b, code_weights):
    f32 = jnp.float32

    _BM = 256
    fulla3 = pl.pallas_call(
        _fulla_body,
        grid=(_BATCH // _BM,),
        in_specs=[
            pl.BlockSpec((_BM, _DIM), lambda i: (i, 0)),
            pl.BlockSpec((_DIM, _NTB * (_DIM - 1)), lambda i: (0, 0)),
            pl.BlockSpec((1, _NTB * (_DIM - 1)), lambda i: (0, 0)),
        ],
        out_specs=pl.BlockSpec((_BM, _NTB, _DIM), lambda i: (i, 0, 0)),
        out_shape=jax.ShapeDtypeStruct((_BATCH, _NTB, _DIM), f32),
    )(features, W, b.reshape(1, _NTB * (_DIM - 1)))
    full_a = fulla3.reshape(_N, _DIM)

    d8 = jnp.broadcast_to(sparse_defaults[None, :], (8, _N))
    logits, part8 = pl.pallas_call(
        _logits_body,
        grid=(_N // _TM, _NUM_CODES // _TN),
        in_specs=[
            pl.BlockSpec((_TM, _DIM), lambda i, j: (i, 0)),
            pl.BlockSpec((_TN, _DIM), lambda i, j: (j, 0)),
            pl.BlockSpec((8, _TM), lambda i, j: (0, i)),
        ],
        out_specs=[
            pl.BlockSpec((_TM, _TN), lambda i, j: (i, j)),
            pl.BlockSpec((8, _TN), lambda i, j: (i, j)),
        ],
        out_shape=[
            jax.ShapeDtypeStruct((_N, _NUM_CODES), f32),
            jax.ShapeDtypeStruct((8 * (_N // _TM), _NUM_CODES), f32),
        ],
    )(full_a, code_weights, d8)

    r_all = jnp.concatenate([event_indices[:, 0].astype(jnp.int32),
                             jnp.arange(_N, dtype=jnp.int32)])
    c_all = jnp.concatenate([event_indices[:, 1].astype(jnp.int32),
                             sparse_indices.astype(jnp.int32)])
    gev, gsv = _sc_dots(full_a, code_weights, r_all, c_all)
    d_col = sparse_defaults.reshape(_N, 1)
    v_col = sparse_values.reshape(_N, 1)
    m2 = mask.astype(f32).reshape(8, 128)

    loss = pl.pallas_call(
        _final_body,
        in_specs=[
            pl.BlockSpec((8 * (_N // _TM), _NUM_CODES), lambda: (0, 0)),
            pl.BlockSpec((_N, 1), lambda: (0, 0)),
            pl.BlockSpec((_N, 1), lambda: (0, 0)),
            pl.BlockSpec((_N, 16), lambda: (0, 0)),
            pl.BlockSpec((_NW, 16), lambda: (0, 0)),
            pl.BlockSpec((8, 128), lambda: (0, 0)),
        ],
        out_specs=pl.BlockSpec(memory_space=pltpu.SMEM),
        out_shape=jax.ShapeDtypeStruct((1, 1), f32),
    )(part8, d_col, v_col, gsv, gev, m2)

    return loss[0, 0], logits
